# baseline (speedup 1.0000x reference)
import functools

import jax
import jax.numpy as jnp
from jax import lax
from jax.experimental import pallas as pl
from jax.experimental.pallas import tpu as pltpu
from jax.experimental.pallas import tpu_sc as plsc

N_NODES = 10000
N_EDGES = 320000
D = 128

NC = 2
NS = 16
L = 16
NW = NC * NS
EDGES_PER_WORKER = N_EDGES // NW
WIN = 40
NWIN = EDGES_PER_WORKER // WIN
NBUF = 4
ROWS_PER_SUB = 624
TAIL_ROWS = N_NODES - NS * ROWS_PER_SUB


def _sc_aggregate(x, pk, ew, zeros):
    mesh = plsc.VectorSubcoreMesh(core_axis_name="c", subcore_axis_name="s")

    @functools.partial(
        pl.kernel,
        out_type=jax.ShapeDtypeStruct((NC, N_NODES, D), jnp.float32),
        mesh=mesh,
        scratch_types=[
            pltpu.VMEM((EDGES_PER_WORKER,), jnp.int32),
            pltpu.VMEM((EDGES_PER_WORKER,), jnp.float32),
            pltpu.VMEM((NBUF, WIN), jnp.int32),
            pltpu.VMEM((NBUF, WIN), jnp.int32),
            pltpu.VMEM((NBUF, WIN, D), jnp.float32),
            pltpu.VMEM_SHARED((N_NODES, D), jnp.float32),
            pltpu.SemaphoreType.DMA,
            pltpu.SemaphoreType.DMA,
            pltpu.SemaphoreType.DMA,
            pltpu.SemaphoreType.DMA,
        ],
    )
    def k(x_hbm, pk_hbm, ew_hbm, z_hbm, out_hbm,
          pk_v, w_v, si_st, di_st, rows_v, acc_sh,
          gsem0, gsem1, gsem2, gsem3):
        c = lax.axis_index("c")
        s = lax.axis_index("s")
        wid = s * NC + c
        r0 = s * ROWS_PER_SUB
        pltpu.sync_copy(z_hbm.at[pl.ds(r0, ROWS_PER_SUB)],
                        acc_sh.at[pl.ds(r0, ROWS_PER_SUB)])

        @pl.when(s == NS - 1)
        def _():
            t0 = NS * ROWS_PER_SUB
            pltpu.sync_copy(z_hbm.at[pl.ds(t0, TAIL_ROWS)],
                            acc_sh.at[pl.ds(t0, TAIL_ROWS)])

        plsc.subcore_barrier()

        pltpu.sync_copy(pk_hbm.at[wid], pk_v)
        pltpu.sync_copy(ew_hbm.at[wid], w_v)

        gsems = (gsem0, gsem1, gsem2, gsem3)
        mask = jnp.full((L,), 0xFFFF, jnp.int32)

        def unpack(g, b):
            for off in (0, 16, 24):
                p = pk_v[pl.ds(g * WIN + off, L)]
                si_st[b, pl.ds(off, L)] = p & mask
                di_st[b, pl.ds(off, L)] = p >> 16

        def start_gather(g, b):
            unpack(g, b)
            pltpu.make_async_copy(x_hbm.at[si_st.at[b]],
                                  rows_v.at[b], gsems[b]).start()

        def wait_gather(b):
            pltpu.make_async_copy(x_hbm.at[si_st.at[b]],
                                  rows_v.at[b], gsems[b]).wait()

        def scale(g, b):
            @plsc.parallel_loop(0, WIN, step=L, unroll=2)
            def _(c0):
                wv = w_v[pl.ds(g * WIN + c0, L)]
                for i in range(L):
                    bc = lax.broadcast_in_dim(wv[i], (L,), ())
                    for j in range(D // L):
                        sl = pl.ds(j * L, L)
                        rows_v[b, c0 + i, sl] = rows_v[b, c0 + i, sl] * bc

        def scatter(b):
            pltpu.sync_copy(rows_v.at[b], acc_sh.at[di_st.at[b]], add=True)

        start_gather(0, 0)
        start_gather(1, 1)
        start_gather(2, 2)

        @pl.loop(0, NWIN // NBUF)
        def _(k4):
            for b in range(NBUF):
                g = NBUF * k4 + b
                wait_gather(b)
                scale(g, b)
                scatter(b)

                @pl.when(g + 3 < NWIN)
                def _():
                    start_gather(g + 3, (b + 3) % NBUF)

        for b in range(NWIN % NBUF):
            g = (NWIN // NBUF) * NBUF + b
            wait_gather(b)
            scale(g, b)
            scatter(b)

        plsc.subcore_barrier()
        pltpu.sync_copy(acc_sh.at[pl.ds(r0, ROWS_PER_SUB)],
                        out_hbm.at[c].at[pl.ds(r0, ROWS_PER_SUB)])

        @pl.when(s == NS - 1)
        def _():
            t0 = NS * ROWS_PER_SUB
            pltpu.sync_copy(acc_sh.at[pl.ds(t0, TAIL_ROWS)],
                            out_hbm.at[c].at[pl.ds(t0, TAIL_ROWS)])

    return k(x, pk, ew, zeros)


def _linear(p, wt, b2):
    R = 1000

    def body(p_ref, wt_ref, b_ref, o_ref):
        acc = p_ref[0] + p_ref[1]
        o_ref[...] = (
            jnp.dot(acc.astype(jnp.bfloat16), wt_ref[...].astype(jnp.bfloat16),
                    preferred_element_type=jnp.float32)
            + b_ref[...]
        )

    return pl.pallas_call(
        body,
        grid=(N_NODES // R,),
        in_specs=[
            pl.BlockSpec((NC, R, D), lambda i: (0, i, 0)),
            pl.BlockSpec((D, D), lambda i: (0, 0)),
            pl.BlockSpec((1, D), lambda i: (0, 0)),
        ],
        out_specs=pl.BlockSpec((R, D), lambda i: (i, 0)),
        out_shape=jax.ShapeDtypeStruct((N_NODES, D), jnp.float32),
    )(p, wt, b2)


def kernel(x, edge_index, edge_weight, W, b):
    src = edge_index[0].astype(jnp.int32)
    dst = edge_index[1].astype(jnp.int32)
    pk = (src | (dst << 16)).reshape(NW, EDGES_PER_WORKER)
    ew = edge_weight.reshape(NW, EDGES_PER_WORKER)
    zeros = jnp.zeros((N_NODES, D), jnp.float32)
    p = _sc_aggregate(x, pk, ew, zeros)
    return _linear(p, W.T, b.reshape(1, D))


# speedup vs baseline: 1.3930x; 1.3930x over previous
import functools

import jax
import jax.numpy as jnp
from jax import lax
from jax.experimental import pallas as pl
from jax.experimental.pallas import tpu as pltpu
from jax.experimental.pallas import tpu_sc as plsc

N_NODES = 10000
N_EDGES = 320000
D = 128

NC = 2
NS = 16
L = 16
NW = NC * NS
EDGES_PER_WORKER = N_EDGES // NW
WIN = 80
NWIN = EDGES_PER_WORKER // WIN
ROWS_PER_SUB = 624
TAIL_ROWS = N_NODES - NS * ROWS_PER_SUB


def _sc_aggregate(x, src, dst, ew, zeros):
    mesh = plsc.VectorSubcoreMesh(core_axis_name="c", subcore_axis_name="s")

    @functools.partial(
        pl.kernel,
        out_type=jax.ShapeDtypeStruct((NC, N_NODES, D), jnp.float32),
        mesh=mesh,
        scratch_types=[
            pltpu.VMEM((EDGES_PER_WORKER,), jnp.int32),
            pltpu.VMEM((EDGES_PER_WORKER,), jnp.int32),
            pltpu.VMEM((EDGES_PER_WORKER,), jnp.float32),
            pltpu.VMEM((2, WIN), jnp.int32),
            pltpu.VMEM((2, WIN, D), jnp.float32),
            pltpu.VMEM_SHARED((N_NODES, D), jnp.float32),
            pltpu.SemaphoreType.DMA,
            pltpu.SemaphoreType.DMA,
        ],
    )
    def k(x_hbm, src_hbm, dst_hbm, ew_hbm, z_hbm, out_hbm,
          si_v, di_v, w_v, di_st, rows_v, acc_sh, gsem0, gsem1):
        c = lax.axis_index("c")
        s = lax.axis_index("s")
        wid = s * NC + c
        r0 = s * ROWS_PER_SUB
        pltpu.sync_copy(z_hbm.at[pl.ds(r0, ROWS_PER_SUB)],
                        acc_sh.at[pl.ds(r0, ROWS_PER_SUB)])

        @pl.when(s == NS - 1)
        def _():
            t0 = NS * ROWS_PER_SUB
            pltpu.sync_copy(z_hbm.at[pl.ds(t0, TAIL_ROWS)],
                            acc_sh.at[pl.ds(t0, TAIL_ROWS)])

        plsc.subcore_barrier()

        pltpu.sync_copy(src_hbm.at[wid], si_v)
        pltpu.sync_copy(dst_hbm.at[wid], di_v)
        pltpu.sync_copy(ew_hbm.at[wid], w_v)

        gsems = (gsem0, gsem1)

        def start_gather(g, b):
            pltpu.make_async_copy(x_hbm.at[si_v.at[pl.ds(g * WIN, WIN)]],
                                  rows_v.at[b], gsems[b]).start()

        def wait_gather(g, b):
            pltpu.make_async_copy(x_hbm.at[si_v.at[pl.ds(g * WIN, WIN)]],
                                  rows_v.at[b], gsems[b]).wait()

        def scale(g, b):
            @plsc.parallel_loop(0, WIN, step=L, unroll=2)
            def _(c0):
                wv = w_v[pl.ds(g * WIN + c0, L)]
                di_st[b, pl.ds(c0, L)] = di_v[pl.ds(g * WIN + c0, L)]
                for i in range(L):
                    bc = lax.broadcast_in_dim(wv[i], (L,), ())
                    for j in range(D // L):
                        sl = pl.ds(j * L, L)
                        rows_v[b, c0 + i, sl] = rows_v[b, c0 + i, sl] * bc

        def scatter(g, b):
            pltpu.sync_copy(rows_v.at[b], acc_sh.at[di_st.at[b]], add=True)

        start_gather(0, 0)
        start_gather(1, 1)

        @pl.loop(0, (NWIN - 1) // 2)
        def _(k2):
            for b in range(2):
                g = 2 * k2 + b
                wait_gather(g, b)
                scale(g, b)
                scatter(g, b)

                @pl.when(g + 2 < NWIN)
                def _():
                    start_gather(g + 2, b)

        wait_gather(NWIN - 1, 0)
        scale(NWIN - 1, 0)
        scatter(NWIN - 1, 0)

        plsc.subcore_barrier()
        pltpu.sync_copy(acc_sh.at[pl.ds(r0, ROWS_PER_SUB)],
                        out_hbm.at[c].at[pl.ds(r0, ROWS_PER_SUB)])

        @pl.when(s == NS - 1)
        def _():
            t0 = NS * ROWS_PER_SUB
            pltpu.sync_copy(acc_sh.at[pl.ds(t0, TAIL_ROWS)],
                            out_hbm.at[c].at[pl.ds(t0, TAIL_ROWS)])

    return k(x, src, dst, ew, zeros)


def _linear(p, wt, b2):
    R = 1000

    def body(p_ref, wt_ref, b_ref, o_ref):
        acc = p_ref[0] + p_ref[1]
        o_ref[...] = (
            jnp.dot(acc, wt_ref[...], preferred_element_type=jnp.float32)
            + b_ref[...]
        )

    return pl.pallas_call(
        body,
        grid=(N_NODES // R,),
        in_specs=[
            pl.BlockSpec((NC, R, D), lambda i: (0, i, 0)),
            pl.BlockSpec((D, D), lambda i: (0, 0)),
            pl.BlockSpec((1, D), lambda i: (0, 0)),
        ],
        out_specs=pl.BlockSpec((R, D), lambda i: (i, 0)),
        out_shape=jax.ShapeDtypeStruct((N_NODES, D), jnp.float32),
    )(p, wt, b2)


def kernel(x, edge_index, edge_weight, W, b):
    src = edge_index[0].astype(jnp.int32).reshape(NW, EDGES_PER_WORKER)
    dst = edge_index[1].astype(jnp.int32).reshape(NW, EDGES_PER_WORKER)
    ew = edge_weight.reshape(NW, EDGES_PER_WORKER)
    zeros = jnp.zeros((N_NODES, D), jnp.float32)
    p = _sc_aggregate(x, src, dst, ew, zeros)
    return _linear(p, W.T, b.reshape(1, D))
